# baseline (speedup 1.0000x reference)
import jax
import jax.numpy as jnp
from jax import lax
from jax.experimental import pallas as pl
from jax.experimental.pallas import tpu as pltpu
from jax.experimental.pallas import tpu_sc as plsc

_K = 256
_N = 32768
_B = 128
_NC = 2
_NS = 16
_NW = _NC * _NS
_ROWS_PER_W = _B // _NW
_LANES = 16
_CHUNKS = _N // _LANES
_UNROLL = 8
_HI = 0x7FFFFFFF
_DUMP = _N


def _zero_hist(hist_v):
    @plsc.parallel_loop(0, 256, unroll=_UNROLL)
    def _zero(i):
        hist_v[pl.ds(i * _LANES, _LANES)] = jnp.zeros((_LANES,), jnp.int32)


def _level_scan(hist_v, kk):
    def mbody(i, accs):
        return tuple(
            acc + hist_v[pl.ds((j * 16 + i) * _LANES, _LANES)]
            for j, acc in enumerate(accs)
        )

    g = lax.fori_loop(
        0, 16, mbody,
        tuple(jnp.zeros((_LANES,), jnp.int32) for _ in range(16)),
    )

    gt = [jnp.sum(gj) for gj in g]

    zero = jnp.int32(0)
    s_after = zero
    sel_j = zero
    sel_after = zero
    for j in reversed(range(16)):
        crossing = jnp.logical_and(s_after < kk, s_after + gt[j] >= kk)
        sel_j = jnp.where(crossing, jnp.int32(j), sel_j)
        sel_after = jnp.where(crossing, s_after, sel_after)
        s_after = s_after + gt[j]

    base = sel_j * (16 * _LANES)
    tb = [jnp.sum(hist_v[pl.ds(base + i * _LANES, _LANES)]) for i in range(16)]
    s_after2 = sel_after
    sel_i = zero
    count_above = zero
    h_at = zero
    for i in reversed(range(16)):
        crossing = jnp.logical_and(s_after2 < kk, s_after2 + tb[i] >= kk)
        sel_i = jnp.where(crossing, jnp.int32(i), sel_i)
        count_above = jnp.where(crossing, s_after2, count_above)
        h_at = jnp.where(crossing, tb[i], h_at)
        s_after2 = s_after2 + tb[i]

    b = sel_j * 16 + sel_i
    return b, count_above, h_at


def _select_and_mask_row(row_v, cand_a, cand_b, hist_v):
    lane_ids = lax.iota(jnp.int32, _LANES)
    ones = jnp.ones((_LANES,), jnp.int32)
    zeros_i = jnp.zeros((_LANES,), jnp.int32)
    hi_mask = jnp.int32(_HI)
    kk = jnp.int32(_K)

    with jax.named_scope("zero1"):
        _zero_hist(hist_v)

    with jax.named_scope("hist1"):
        @plsc.parallel_loop(0, _CHUNKS, unroll=_UNROLL)
        def _hist1(i):
            xv = row_v[pl.ds(i * _LANES, _LANES)]
            u = lax.bitcast_convert_type(xv, jnp.int32) & hi_mask
            b = u >> 23
            plsc.addupdate_scatter(hist_v, [b * _LANES + lane_ids], ones)

    with jax.named_scope("scan1"):
        b1, ca1, m1 = _level_scan(hist_v, kk)
    kk = kk - ca1
    b1_vec = jnp.full((_LANES,), b1, jnp.int32)

    last_lane = jnp.full((_LANES,), _LANES - 1, jnp.int32)
    with jax.named_scope("compact1"):
        @plsc.parallel_loop(0, _CHUNKS, unroll=_UNROLL,
                            carry=jnp.zeros((_LANES,), jnp.int32))
        def _compact1(i, cnt):
            xv = row_v[pl.ds(i * _LANES, _LANES)]
            u = lax.bitcast_convert_type(xv, jnp.int32) & hi_mask
            eq = (u >> 23) == b1_vec
            eqi = eq.astype(jnp.int32)
            prefix = plsc.cumsum(eqi)
            off = jnp.where(eq, cnt + prefix - 1, _DUMP + lane_ids)
            plsc.store_scatter(cand_a, [off], u)
            return cnt + prefix.at[last_lane].get(mode="promise_in_bounds")

    def _small_level(src, dst, m, kk, shift, bmask, compact):
        n = (m + 15) >> 4
        m_vec = jnp.full((_LANES,), m, jnp.int32)
        _zero_hist(hist_v)

        @plsc.parallel_loop(0, n, unroll=2, carry=jnp.int32(0))
        def _histS(i, c):
            u = src[pl.ds(i * _LANES, _LANES)]
            live = (i * _LANES + lane_ids) < m_vec
            b = (u >> shift) & bmask
            vals = jnp.where(live, ones, zeros_i)
            plsc.addupdate_scatter(hist_v, [b * _LANES + lane_ids], vals)
            return c

        b, ca, h_at = _level_scan(hist_v, kk)
        if compact:
            b_vec = jnp.full((_LANES,), b, jnp.int32)
            last_lane = jnp.full((_LANES,), _LANES - 1, jnp.int32)

            @plsc.parallel_loop(0, n, unroll=2,
                                carry=jnp.zeros((_LANES,), jnp.int32))
            def _compactS(i, cnt):
                u = src[pl.ds(i * _LANES, _LANES)]
                live = (i * _LANES + lane_ids) < m_vec
                eq = jnp.logical_and(((u >> shift) & bmask) == b_vec, live)
                eqi = eq.astype(jnp.int32)
                prefix = plsc.cumsum(eqi)
                off = jnp.where(eq, cnt + prefix - 1, _DUMP + lane_ids)
                plsc.store_scatter(dst, [off], u)
                return cnt + prefix.at[last_lane].get(
                    mode="promise_in_bounds")

        return b, kk - ca, h_at

    bm8 = jnp.int32(0xFF)
    with jax.named_scope("lvl2"):
        b2, kk, m2 = _small_level(cand_a, cand_b, m1, kk, 15, bm8, True)
    with jax.named_scope("lvl3"):
        b3, kk, m3 = _small_level(cand_b, cand_a, m2, kk, 7, bm8, True)
    with jax.named_scope("lvl4"):
        b4, kk4, h_at = _small_level(cand_a, cand_b, m3, kk, 0,
                                     jnp.int32(0x7F), False)

    t = (((((b1 << 8) | b2) << 8) | b3) << 7) | b4
    e = h_at - kk4

    tv = jnp.full((_LANES,), t, jnp.int32)
    zf = jnp.zeros((_LANES,), jnp.float32)

    with jax.named_scope("outpass"):
        @plsc.parallel_loop(0, _CHUNKS, unroll=_UNROLL)
        def _out(i):
            sl = pl.ds(i * _LANES, _LANES)
            xv = row_v[sl]
            u = lax.bitcast_convert_type(xv, jnp.int32) & hi_mask
            row_v[sl] = jnp.where(u >= tv, xv, zf)

    @pl.when(e > 0)
    def _fixup():
        e_vec = jnp.full((_LANES,), e, jnp.int32)

        def fbody(i, cnt):
            sl = pl.ds(i * _LANES, _LANES)
            xv = row_v[sl]
            u = lax.bitcast_convert_type(xv, jnp.int32) & hi_mask
            eq = u == tv
            inc = plsc.cumsum(eq.astype(jnp.int32))
            occ = cnt + inc
            row_v[sl] = jnp.where(
                jnp.logical_and(eq, occ <= e_vec), zf, xv
            )
            return cnt + jnp.max(inc)

        lax.fori_loop(0, _CHUNKS, fbody, jnp.int32(0))


_ROWS_C0 = 3
_ROWS_C1 = 5


def _sc_body(x_hbm, out_hbm, row_v, cand_a, cand_b, hist_v):
    c = lax.axis_index("c")
    s = lax.axis_index("s")
    is_c0 = c == 0
    row0 = jnp.where(is_c0, s * _ROWS_C0, _NS * _ROWS_C0 + s * _ROWS_C1)
    nrows = jnp.where(is_c0, _ROWS_C0, _ROWS_C1)

    def rbody(rr, carry):
        row = row0 + rr
        with jax.named_scope("dma_in"):
            pltpu.sync_copy(x_hbm.at[row], row_v)
        _select_and_mask_row(row_v, cand_a, cand_b, hist_v)
        with jax.named_scope("dma_out"):
            pltpu.sync_copy(row_v, out_hbm.at[row])
        return carry

    lax.fori_loop(0, nrows, rbody, 0)


def kernel(x):
    mesh = plsc.VectorSubcoreMesh(core_axis_name="c", subcore_axis_name="s")
    f = pl.kernel(
        _sc_body,
        out_type=jax.ShapeDtypeStruct((_B, _N), jnp.float32),
        mesh=mesh,
        compiler_params=pltpu.CompilerParams(needs_layout_passes=False),
        scratch_types=[
            pltpu.VMEM((_N,), jnp.float32),
            pltpu.VMEM((_N + _LANES,), jnp.int32),
            pltpu.VMEM((_N + _LANES,), jnp.int32),
            pltpu.VMEM((_LANES * 256,), jnp.int32),
        ],
    )
    return f(x)


# speedup vs baseline: 1.1263x; 1.1263x over previous
import contextlib
import jax
import jax.numpy as jnp
from jax import lax
from jax.experimental import pallas as pl
from jax.experimental.pallas import tpu as pltpu
from jax.experimental.pallas import tpu_sc as plsc

_K = 256
_N = 32768
_B = 128
_NC = 2
_NS = 16
_NW = _NC * _NS
_ROWS_PER_W = _B // _NW
_LANES = 16
_CHUNKS = _N // _LANES
_UNROLL = 8
_HI = 0x7FFFFFFF
_DUMP = _N


def _zero_hist(hist_v):
    @plsc.parallel_loop(0, 256, unroll=_UNROLL)
    def _zero(i):
        hist_v[pl.ds(i * _LANES, _LANES)] = jnp.zeros((_LANES,), jnp.int32)


def _level_scan(hist_v, kk):
    def mbody(i, accs):
        return tuple(
            acc + hist_v[pl.ds((j * 16 + i) * _LANES, _LANES)]
            for j, acc in enumerate(accs)
        )

    g = lax.fori_loop(
        0, 16, mbody,
        tuple(jnp.zeros((_LANES,), jnp.int32) for _ in range(16)),
    )

    gt = [jnp.sum(gj) for gj in g]

    zero = jnp.int32(0)
    s_after = zero
    sel_j = zero
    sel_after = zero
    for j in reversed(range(16)):
        crossing = jnp.logical_and(s_after < kk, s_after + gt[j] >= kk)
        sel_j = jnp.where(crossing, jnp.int32(j), sel_j)
        sel_after = jnp.where(crossing, s_after, sel_after)
        s_after = s_after + gt[j]

    base = sel_j * (16 * _LANES)
    tb = [jnp.sum(hist_v[pl.ds(base + i * _LANES, _LANES)]) for i in range(16)]
    s_after2 = sel_after
    sel_i = zero
    count_above = zero
    h_at = zero
    for i in reversed(range(16)):
        crossing = jnp.logical_and(s_after2 < kk, s_after2 + tb[i] >= kk)
        sel_i = jnp.where(crossing, jnp.int32(i), sel_i)
        count_above = jnp.where(crossing, s_after2, count_above)
        h_at = jnp.where(crossing, tb[i], h_at)
        s_after2 = s_after2 + tb[i]

    b = sel_j * 16 + sel_i
    return b, count_above, h_at


def _select_and_mask_row(row_v, cand_a, cand_b, hist_v):
    lane_ids = lax.iota(jnp.int32, _LANES)
    ones = jnp.ones((_LANES,), jnp.int32)
    zeros_i = jnp.zeros((_LANES,), jnp.int32)
    hi_mask = jnp.int32(_HI)
    kk = jnp.int32(_K)

    with contextlib.nullcontext("zero1"):
        _zero_hist(hist_v)

    with contextlib.nullcontext("hist1"):
        @plsc.parallel_loop(0, _CHUNKS, unroll=_UNROLL)
        def _hist1(i):
            xv = row_v[pl.ds(i * _LANES, _LANES)]
            u = lax.bitcast_convert_type(xv, jnp.int32) & hi_mask
            b = u >> 23
            plsc.addupdate_scatter(hist_v, [b * _LANES + lane_ids], ones)

    with contextlib.nullcontext("scan1"):
        b1, ca1, m1 = _level_scan(hist_v, kk)
    kk = kk - ca1
    b1_vec = jnp.full((_LANES,), b1, jnp.int32)

    last_lane = jnp.full((_LANES,), _LANES - 1, jnp.int32)
    with contextlib.nullcontext("compact1"):
        @plsc.parallel_loop(0, _CHUNKS, unroll=_UNROLL,
                            carry=jnp.zeros((_LANES,), jnp.int32))
        def _compact1(i, cnt):
            xv = row_v[pl.ds(i * _LANES, _LANES)]
            u = lax.bitcast_convert_type(xv, jnp.int32) & hi_mask
            eq = (u >> 23) == b1_vec
            eqi = eq.astype(jnp.int32)
            prefix = plsc.cumsum(eqi)
            off = jnp.where(eq, cnt + prefix - 1, _DUMP + lane_ids)
            plsc.store_scatter(cand_a, [off], u)
            return cnt + prefix.at[last_lane].get(mode="promise_in_bounds")

    def _small_level(src, dst, m, kk, shift, bmask, compact):
        n = (m + 15) >> 4
        m_vec = jnp.full((_LANES,), m, jnp.int32)
        _zero_hist(hist_v)

        @plsc.parallel_loop(0, n, unroll=2, carry=jnp.int32(0))
        def _histS(i, c):
            u = src[pl.ds(i * _LANES, _LANES)]
            live = (i * _LANES + lane_ids) < m_vec
            b = (u >> shift) & bmask
            vals = jnp.where(live, ones, zeros_i)
            plsc.addupdate_scatter(hist_v, [b * _LANES + lane_ids], vals)
            return c

        b, ca, h_at = _level_scan(hist_v, kk)
        if compact:
            b_vec = jnp.full((_LANES,), b, jnp.int32)
            last_lane = jnp.full((_LANES,), _LANES - 1, jnp.int32)

            @plsc.parallel_loop(0, n, unroll=2,
                                carry=jnp.zeros((_LANES,), jnp.int32))
            def _compactS(i, cnt):
                u = src[pl.ds(i * _LANES, _LANES)]
                live = (i * _LANES + lane_ids) < m_vec
                eq = jnp.logical_and(((u >> shift) & bmask) == b_vec, live)
                eqi = eq.astype(jnp.int32)
                prefix = plsc.cumsum(eqi)
                off = jnp.where(eq, cnt + prefix - 1, _DUMP + lane_ids)
                plsc.store_scatter(dst, [off], u)
                return cnt + prefix.at[last_lane].get(
                    mode="promise_in_bounds")

        return b, kk - ca, h_at

    bm8 = jnp.int32(0xFF)
    with contextlib.nullcontext("lvl2"):
        b2, kk, m2 = _small_level(cand_a, cand_b, m1, kk, 15, bm8, True)
    with contextlib.nullcontext("lvl3"):
        b3, kk, m3 = _small_level(cand_b, cand_a, m2, kk, 7, bm8, True)
    with contextlib.nullcontext("lvl4"):
        b4, kk4, h_at = _small_level(cand_a, cand_b, m3, kk, 0,
                                     jnp.int32(0x7F), False)

    t = (((((b1 << 8) | b2) << 8) | b3) << 7) | b4
    e = h_at - kk4

    tv = jnp.full((_LANES,), t, jnp.int32)
    zf = jnp.zeros((_LANES,), jnp.float32)

    with contextlib.nullcontext("outpass"):
        @plsc.parallel_loop(0, _CHUNKS, unroll=_UNROLL)
        def _out(i):
            sl = pl.ds(i * _LANES, _LANES)
            xv = row_v[sl]
            u = lax.bitcast_convert_type(xv, jnp.int32) & hi_mask
            row_v[sl] = jnp.where(u >= tv, xv, zf)

    @pl.when(e > 0)
    def _fixup():
        e_vec = jnp.full((_LANES,), e, jnp.int32)

        def fbody(i, cnt):
            sl = pl.ds(i * _LANES, _LANES)
            xv = row_v[sl]
            u = lax.bitcast_convert_type(xv, jnp.int32) & hi_mask
            eq = u == tv
            inc = plsc.cumsum(eq.astype(jnp.int32))
            occ = cnt + inc
            row_v[sl] = jnp.where(
                jnp.logical_and(eq, occ <= e_vec), zf, xv
            )
            return cnt + jnp.max(inc)

        lax.fori_loop(0, _CHUNKS, fbody, jnp.int32(0))


_ROWS_C0 = 4
_ROWS_C1 = 4


def _sc_body(x_hbm, out_hbm, row_v, cand_a, cand_b, hist_v):
    c = lax.axis_index("c")
    s = lax.axis_index("s")
    is_c0 = c == 0
    row0 = jnp.where(is_c0, s * _ROWS_C0, _NS * _ROWS_C0 + s * _ROWS_C1)
    nrows = jnp.where(is_c0, _ROWS_C0, _ROWS_C1)

    def rbody(rr, carry):
        row = row0 + rr
        with contextlib.nullcontext("dma_in"):
            pltpu.sync_copy(x_hbm.at[row], row_v)
        _select_and_mask_row(row_v, cand_a, cand_b, hist_v)
        with contextlib.nullcontext("dma_out"):
            pltpu.sync_copy(row_v, out_hbm.at[row])
        return carry

    lax.fori_loop(0, nrows, rbody, 0)


def kernel(x):
    mesh = plsc.VectorSubcoreMesh(core_axis_name="c", subcore_axis_name="s")
    f = pl.kernel(
        _sc_body,
        out_type=jax.ShapeDtypeStruct((_B, _N), jnp.float32),
        mesh=mesh,
        compiler_params=pltpu.CompilerParams(needs_layout_passes=False),
        scratch_types=[
            pltpu.VMEM((_N,), jnp.float32),
            pltpu.VMEM((_N + _LANES,), jnp.int32),
            pltpu.VMEM((_N + _LANES,), jnp.int32),
            pltpu.VMEM((_LANES * 256,), jnp.int32),
        ],
    )
    return f(x)


# speedup vs baseline: 1.5223x; 1.3515x over previous
import contextlib
import jax
import jax.numpy as jnp
from jax import lax
from jax.experimental import pallas as pl
from jax.experimental.pallas import tpu as pltpu
from jax.experimental.pallas import tpu_sc as plsc

_K = 256
_N = 32768
_B = 128
_NC = 2
_NS = 16
_NW = _NC * _NS
_ROWS_PER_W = _B // _NW
_LANES = 16
_CHUNKS = _N // _LANES
_UNROLL = 8
_HI = 0x7FFFFFFF
_DUMP = _N


def _zero_hist(hist_v):
    @plsc.parallel_loop(0, 256, unroll=_UNROLL)
    def _zero(i):
        hist_v[pl.ds(i * _LANES, _LANES)] = jnp.zeros((_LANES,), jnp.int32)


def _level_scan(hist_v, kk):
    def mbody(i, accs):
        return tuple(
            acc + hist_v[pl.ds((j * 16 + i) * _LANES, _LANES)]
            for j, acc in enumerate(accs)
        )

    g = lax.fori_loop(
        0, 16, mbody,
        tuple(jnp.zeros((_LANES,), jnp.int32) for _ in range(16)),
    )

    gt = [jnp.sum(gj) for gj in g]

    zero = jnp.int32(0)
    s_after = zero
    sel_j = zero
    sel_after = zero
    for j in reversed(range(16)):
        crossing = jnp.logical_and(s_after < kk, s_after + gt[j] >= kk)
        sel_j = jnp.where(crossing, jnp.int32(j), sel_j)
        sel_after = jnp.where(crossing, s_after, sel_after)
        s_after = s_after + gt[j]

    base = sel_j * (16 * _LANES)
    tb = [jnp.sum(hist_v[pl.ds(base + i * _LANES, _LANES)]) for i in range(16)]
    s_after2 = sel_after
    sel_i = zero
    count_above = zero
    h_at = zero
    for i in reversed(range(16)):
        crossing = jnp.logical_and(s_after2 < kk, s_after2 + tb[i] >= kk)
        sel_i = jnp.where(crossing, jnp.int32(i), sel_i)
        count_above = jnp.where(crossing, s_after2, count_above)
        h_at = jnp.where(crossing, tb[i], h_at)
        s_after2 = s_after2 + tb[i]

    b = sel_j * 16 + sel_i
    return b, count_above, h_at


def _select_and_mask_row(row_v, cand_a, hist_v):
    lane_ids = lax.iota(jnp.int32, _LANES)
    ones = jnp.ones((_LANES,), jnp.int32)
    zeros_i = jnp.zeros((_LANES,), jnp.int32)
    hi_mask = jnp.int32(_HI)
    kk = jnp.int32(_K)

    with contextlib.nullcontext("zero1"):
        _zero_hist(hist_v)

    with contextlib.nullcontext("hist1"):
        @plsc.parallel_loop(0, _CHUNKS, unroll=_UNROLL)
        def _hist1(i):
            xv = row_v[pl.ds(i * _LANES, _LANES)]
            u = lax.bitcast_convert_type(xv, jnp.int32) & hi_mask
            b = u >> 23
            plsc.addupdate_scatter(hist_v, [b * _LANES + lane_ids], ones)

    with contextlib.nullcontext("scan1"):
        b1, ca1, m1 = _level_scan(hist_v, kk)
    kk = kk - ca1
    b1_vec = jnp.full((_LANES,), b1, jnp.int32)

    zf = jnp.zeros((_LANES,), jnp.float32)
    last_lane = jnp.full((_LANES,), _LANES - 1, jnp.int32)
    with contextlib.nullcontext("compact1"):
        @plsc.parallel_loop(0, _CHUNKS, unroll=_UNROLL,
                            carry=jnp.zeros((_LANES,), jnp.int32))
        def _compact1(i, cnt):
            sl = pl.ds(i * _LANES, _LANES)
            xv = row_v[sl]
            u = lax.bitcast_convert_type(xv, jnp.int32) & hi_mask
            b = u >> 23
            is_cand = b == b1_vec
            row_v[sl] = jnp.where(b >= b1_vec, xv, zf)
            eqi = is_cand.astype(jnp.int32)
            prefix = plsc.cumsum(eqi)
            off = jnp.where(is_cand, cnt + prefix - 1, _DUMP + lane_ids)
            plsc.store_scatter(cand_a, [off], i * _LANES + lane_ids)
            return cnt + prefix.at[last_lane].get(mode="promise_in_bounds")

    m1_vec = jnp.full((_LANES,), m1, jnp.int32)
    n1 = (m1 + 15) >> 4

    def _small_level(kk, prefix_p, pshift, shift, bmask):
        p_vec = jnp.full((_LANES,), prefix_p, jnp.int32)
        _zero_hist(hist_v)

        @plsc.parallel_loop(0, n1, unroll=2, carry=jnp.int32(0))
        def _histS(i, c):
            live = (i * _LANES + lane_ids) < m1_vec
            pos = jnp.where(live, cand_a[pl.ds(i * _LANES, _LANES)], zeros_i)
            xg = plsc.load_gather(row_v, [pos])
            u = lax.bitcast_convert_type(xg, jnp.int32) & hi_mask
            ok = jnp.logical_and(live, (u >> pshift) == p_vec)
            b = (u >> shift) & bmask
            vals = jnp.where(ok, ones, zeros_i)
            plsc.addupdate_scatter(hist_v, [b * _LANES + lane_ids], vals)
            return c

        b, ca, h_at = _level_scan(hist_v, kk)
        return b, kk - ca, h_at

    bm8 = jnp.int32(0xFF)
    with contextlib.nullcontext("lvl2"):
        b2, kk, _ = _small_level(kk, b1, 23, 15, bm8)
        p2 = (b1 << 8) | b2
    with contextlib.nullcontext("lvl3"):
        b3, kk, _ = _small_level(kk, p2, 15, 7, bm8)
        p3 = (p2 << 8) | b3
    with contextlib.nullcontext("lvl4"):
        b4, kk4, h_at = _small_level(kk, p3, 7, 0, jnp.int32(0x7F))

    t = (p3 << 7) | b4
    e = h_at - kk4

    tv = jnp.full((_LANES,), t, jnp.int32)

    e_vec = jnp.full((_LANES,), e, jnp.int32)
    with contextlib.nullcontext("fixcand"):
        @plsc.parallel_loop(0, n1, unroll=2,
                            carry=jnp.zeros((_LANES,), jnp.int32))
        def _fixc(i, cnt):
            live = (i * _LANES + lane_ids) < m1_vec
            pos = jnp.where(live, cand_a[pl.ds(i * _LANES, _LANES)], zeros_i)
            xg = plsc.load_gather(row_v, [pos])
            u = lax.bitcast_convert_type(xg, jnp.int32) & hi_mask
            eq = u == tv
            occ = cnt + plsc.cumsum(jnp.where(live, eq, False)
                                    .astype(jnp.int32))
            zero_it = jnp.logical_or(u < tv, jnp.logical_and(eq, occ <= e_vec))
            newv = jnp.where(zero_it, zf, xg)
            off = jnp.where(live, pos, _DUMP + lane_ids)
            plsc.store_scatter(row_v, [off], newv)
            return occ.at[last_lane].get(mode="promise_in_bounds")


def _sc_body(x_hbm, out_hbm, row_a, row_b, cand_a, hist_v,
             sem_ia, sem_ib, sem_oa, sem_ob):
    wid = lax.axis_index("c") * _NS + lax.axis_index("s")
    base_row = wid * _ROWS_PER_W

    def _in(row, buf, sem):
        return pltpu.make_async_copy(x_hbm.at[row], buf.at[pl.ds(0, _N)], sem)

    def _outc(row, buf, sem):
        return pltpu.make_async_copy(buf.at[pl.ds(0, _N)], out_hbm.at[row], sem)

    _in(base_row, row_a, sem_ia).start()

    def pbody(g, carry):
        r0 = base_row + 2 * g
        r1 = r0 + 1

        @pl.when(g > 0)
        def _drain_b():
            _outc(r1 - 2, row_b, sem_ob).wait()

        _in(r1, row_b, sem_ib).start()
        _in(r0, row_a, sem_ia).wait()
        _select_and_mask_row(row_a, cand_a, hist_v)
        _outc(r0, row_a, sem_oa).start()
        _in(r1, row_b, sem_ib).wait()
        _select_and_mask_row(row_b, cand_a, hist_v)
        _outc(r0, row_a, sem_oa).wait()

        @pl.when(g + 1 < _ROWS_PER_W // 2)
        def _prefetch():
            _in(r0 + 2, row_a, sem_ia).start()

        _outc(r1, row_b, sem_ob).start()
        return carry

    lax.fori_loop(0, _ROWS_PER_W // 2, pbody, 0)
    _outc(base_row + _ROWS_PER_W - 1, row_b, sem_ob).wait()


def kernel(x):
    mesh = plsc.VectorSubcoreMesh(core_axis_name="c", subcore_axis_name="s")
    f = pl.kernel(
        _sc_body,
        out_type=jax.ShapeDtypeStruct((_B, _N), jnp.float32),
        mesh=mesh,
        compiler_params=pltpu.CompilerParams(needs_layout_passes=False),
        scratch_types=[
            pltpu.VMEM((_N + _LANES,), jnp.float32),
            pltpu.VMEM((_N + _LANES,), jnp.float32),
            pltpu.VMEM((_N + _LANES,), jnp.int32),
            pltpu.VMEM((_LANES * 256,), jnp.int32),
            pltpu.SemaphoreType.DMA,
            pltpu.SemaphoreType.DMA,
            pltpu.SemaphoreType.DMA,
            pltpu.SemaphoreType.DMA,
        ],
    )
    return f(x)
